# baseline (speedup 1.0000x reference)
import functools

import jax
import jax.numpy as jnp
from jax import lax
from jax.experimental import pallas as pl
from jax.experimental.pallas import tpu as pltpu
from jax.experimental.pallas import tpu_sc as plsc

_STATIC = 26
_CONT = 13
_ED = 64
_BATCH = 16384
_NPAIR = _STATIC // 2
_OUTW = (_STATIC + _CONT) * _ED

_NC, _NS = 2, 16
_NW = _NC * _NS
_RPW = _BATCH // _NW
_C = 16
_NCHUNK = _RPW // _C


def _sc_body(si_ev_ref, si_od_ref, tab2_ref, cont_ref, out_ref,
             ev_v, od_v, pidx_v, gbuf, cbuf, sem):
    w = lax.axis_index("s") * _NC + lax.axis_index("c")
    base = w * _RPW
    pltpu.sync_copy(si_ev_ref.at[:, pl.ds(base, _RPW)], ev_v)
    pltpu.sync_copy(si_od_ref.at[:, pl.ds(base, _RPW)], od_v)

    def compute_pidx(g, carry):
        r0 = g * 16
        for k in range(_NPAIR):
            ev = ev_v[k, pl.ds(r0, 16)]
            od = od_v[k, pl.ds(r0, 16)]
            pidx_v[pl.ds(k * _RPW + r0, 16)] = ev * _STATIC + od
        return carry

    lax.fori_loop(0, _RPW // 16, compute_pidx, 0)

    def chunk(c, carry):
        r0 = c * _C
        copies = []
        for k in range(_NPAIR):
            copies.append(pltpu.async_copy(
                tab2_ref.at[pidx_v.at[pl.ds(k * _RPW + r0, _C)]],
                gbuf.at[k], sem))
        copies.append(pltpu.async_copy(
            cont_ref.at[pl.ds(base + r0, _C)], cbuf, sem))
        for cp in copies:
            cp.wait()
        copies = []
        for k in range(_NPAIR):
            copies.append(pltpu.async_copy(
                gbuf.at[k],
                out_ref.at[pl.ds(base + r0, _C), pl.ds(128 * k, 128)],
                sem))
        copies.append(pltpu.async_copy(
            cbuf,
            out_ref.at[pl.ds(base + r0, _C), pl.ds(_STATIC * _ED, _CONT * _ED)],
            sem))
        for cp in copies:
            cp.wait()
        return carry

    lax.fori_loop(0, _NCHUNK, chunk, 0)


_sc_gather = functools.partial(
    pl.kernel,
    out_type=jax.ShapeDtypeStruct((_BATCH, _OUTW), jnp.float32),
    mesh=plsc.VectorSubcoreMesh(
        core_axis_name="c", subcore_axis_name="s", num_cores=_NC, num_subcores=_NS),
    scratch_types=[
        pltpu.VMEM((_NPAIR, _RPW), jnp.int32),
        pltpu.VMEM((_NPAIR, _RPW), jnp.int32),
        pltpu.VMEM((_NPAIR * _RPW,), jnp.int32),
        pltpu.VMEM((_NPAIR, _C, 128), jnp.float32),
        pltpu.VMEM((_C, _CONT * _ED), jnp.float32),
        pltpu.SemaphoreType.DMA,
    ],
)(_sc_body)


_R = 256


def _tc_linear_body(ci_ref, wt_ref, b_ref, out_ref):
    out_ref[...] = (
        jnp.dot(ci_ref[...], wt_ref[...], preferred_element_type=jnp.float32)
        + b_ref[...])


def kernel(static_input, continuous_input, table, W, b):
    pr = jnp.arange(_STATIC * _STATIC, dtype=jnp.int32)
    tab2 = jnp.concatenate(
        [table[pr // _STATIC], table[pr % _STATIC]], axis=1)
    si_ev = static_input[:, 0::2].T
    si_od = static_input[:, 1::2].T
    wt = W.T
    b2 = b.reshape(1, _CONT * _ED)

    cont = pl.pallas_call(
        _tc_linear_body,
        grid=(_BATCH // _R,),
        in_specs=[
            pl.BlockSpec((_R, _CONT), lambda i: (i, 0)),
            pl.BlockSpec((_CONT, _CONT * _ED), lambda i: (0, 0)),
            pl.BlockSpec((1, _CONT * _ED), lambda i: (0, 0)),
        ],
        out_specs=pl.BlockSpec((_R, _CONT * _ED), lambda i: (i, 0)),
        out_shape=jax.ShapeDtypeStruct((_BATCH, _CONT * _ED), jnp.float32),
    )(continuous_input, wt, b2)

    out = _sc_gather(si_ev, si_od, tab2, cont)
    return out


# speedup vs baseline: 7.9190x; 7.9190x over previous
import jax
import jax.numpy as jnp
from jax.experimental import pallas as pl
from jax.experimental.pallas import tpu as pltpu

_STATIC = 26
_CONT = 13
_ED = 64
_BATCH = 16384
_OUTW = (_STATIC + _CONT) * _ED
_R = 1024


def _tc_body(sit_ref, cit_ref, tab_ref, wt_ref, b_ref, out_ref, tab2_scr, wtb_scr):
    @pl.when(pl.program_id(0) == 0)
    def _init():
        tab2_scr[...] = jnp.zeros((_ED, 2 * _ED), jnp.float32)
        tab2_scr[0:_STATIC, 0:_ED] = tab_ref[...]
        tab2_scr[32:32 + _STATIC, _ED:2 * _ED] = tab_ref[...]
        wtb_scr[0:_CONT, :] = wt_ref[...]
        wtb_scr[_CONT:, :] = b_ref[...].reshape(1, _CONT * _ED)

    sit = sit_ref[...]
    tab2 = tab2_scr[...]
    sub = jax.lax.broadcasted_iota(jnp.int32, (_ED, _R), 0)
    for p in range(_STATIC // 2):
        r0 = sit[2 * p: 2 * p + 1, :]
        r1 = sit[2 * p + 1: 2 * p + 2, :]
        sel = jnp.where(sub < 32, r0, r1 + 32)
        onehot2 = (sel == sub).astype(jnp.float32)
        out_ref[2 * _ED * p: 2 * _ED * (p + 1), :] = jax.lax.dot_general(
            tab2, onehot2, (((0,), (0,)), ((), ())),
            preferred_element_type=jnp.float32)
    cit1 = jnp.concatenate(
        [cit_ref[...], jnp.ones((1, _R), jnp.float32)], axis=0)
    out_ref[_STATIC * _ED:, :] = jax.lax.dot_general(
        wtb_scr[...], cit1, (((0,), (0,)), ((), ())),
        preferred_element_type=jnp.float32)


def kernel(static_input, continuous_input, table, W, b):
    si_t = static_input.T
    ci_t = continuous_input.T
    wt = W.T

    out_t = pl.pallas_call(
        _tc_body,
        grid=(_BATCH // _R,),
        in_specs=[
            pl.BlockSpec((_STATIC, _R), lambda i: (0, i)),
            pl.BlockSpec((_CONT, _R), lambda i: (0, i)),
            pl.BlockSpec((_STATIC, _ED), lambda i: (0, 0)),
            pl.BlockSpec((_CONT, _CONT * _ED), lambda i: (0, 0)),
            pl.BlockSpec((_CONT * _ED,), lambda i: (0,)),
        ],
        out_specs=pl.BlockSpec((_OUTW, _R), lambda i: (0, i)),
        out_shape=jax.ShapeDtypeStruct((_OUTW, _BATCH), jnp.float32),
        scratch_shapes=[
            pltpu.VMEM((_ED, 2 * _ED), jnp.float32),
            pltpu.VMEM((_CONT + 1, _CONT * _ED), jnp.float32),
        ],
    )(si_t, ci_t, table, wt, b)
    return out_t.T
